# baseline (speedup 1.0000x reference)
import functools

import jax
import jax.numpy as jnp
from jax import lax
from jax.experimental import pallas as pl
from jax.experimental.pallas import tpu as pltpu
from jax.experimental.pallas import tpu_sc as plsc

_NC, _NS = 2, 16
_NW = _NC * _NS
_NB = 4
_D = 2


def kernel(inputs):
    B, H, W, C = inputs.shape
    WW = W // 2
    n_img_rows = B * H
    rows_w = n_img_rows // _NW

    idx_tab = jnp.stack([jnp.arange(1, W, 2, dtype=jnp.int32),
                         jnp.arange(0, W, 2, dtype=jnp.int32)])

    mesh = plsc.VectorSubcoreMesh(
        core_axis_name="c", subcore_axis_name="s",
        num_cores=_NC, num_subcores=_NS,
    )

    @functools.partial(
        pl.kernel,
        out_type=jax.ShapeDtypeStruct((B, H * WW, C), inputs.dtype),
        mesh=mesh,
        scratch_types=[
            pltpu.VMEM((2, WW), jnp.int32),
            *[pltpu.VMEM((WW, C), inputs.dtype) for _ in range(_NB)],
            *[pltpu.SemaphoreType.DMA for _ in range(2 * _NB)],
        ],
        compiler_params=pltpu.CompilerParams(use_tc_tiling_on_sc=False),
    )
    def seq_refresh(x_hbm, idx_hbm, o_hbm, idx_v, *rest):
        bufs = rest[:_NB]
        sin = rest[_NB:2 * _NB]
        sout = rest[2 * _NB:]
        wid = lax.axis_index("s") * _NC + lax.axis_index("c")

        pltpu.sync_copy(idx_hbm, idx_v)

        def bh(u):
            j = wid * rows_w + u
            return j // H, j % H

        def start_gather(u, k):
            b, h = bh(u)
            return pltpu.async_copy(
                x_hbm.at[b, h].at[idx_v.at[h % 2]], bufs[k], sin[k])

        def start_put(u, k):
            b, h = bh(u)
            return pltpu.async_copy(
                bufs[k], o_hbm.at[b, pl.ds(h * WW, WW), :], sout[k])

        g = [None] * rows_w
        ocp = [None] * rows_w
        for u in range(rows_w):
            k = u % _NB
            if u >= _NB:
                ocp[u - _NB].wait()
            g[u] = start_gather(u, k)
            ud = u - _D
            if ud >= 0:
                g[ud].wait()
                ocp[ud] = start_put(ud, ud % _NB)
        for u in range(rows_w - _D, rows_w):
            g[u].wait()
            ocp[u] = start_put(u, u % _NB)
        for u in range(rows_w - _NB, rows_w):
            ocp[u].wait()

    return seq_refresh(inputs, idx_tab)


# speedup vs baseline: 1.2914x; 1.2914x over previous
import functools

import jax
import jax.numpy as jnp
from jax import lax
from jax.experimental import pallas as pl
from jax.experimental.pallas import tpu as pltpu
from jax.experimental.pallas import tpu_sc as plsc

_NC, _NS = 2, 16
_NW = _NC * _NS
_CS = 48
_L = 16


def kernel(inputs):
    B, H, W, C = inputs.shape
    WW = W // 2
    xt = inputs.transpose(0, 1, 3, 2)
    n_units = B * (H // 2) * (C // _CS)
    units_w = n_units // _NW
    n_j = WW // _L

    cols_tab = (2 * jnp.arange(WW, dtype=jnp.int32)[None, :]
                + jnp.arange(2, dtype=jnp.int32)[:, None])

    mesh = plsc.VectorSubcoreMesh(
        core_axis_name="c", subcore_axis_name="s",
        num_cores=_NC, num_subcores=_NS,
    )

    @functools.partial(
        pl.kernel,
        out_type=jax.ShapeDtypeStruct((B, C, H * WW), inputs.dtype),
        mesh=mesh,
        scratch_types=[
            pltpu.VMEM((2, WW), jnp.int32),
            *[pltpu.VMEM((_CS, W), inputs.dtype) for _ in range(6)],
            *[pltpu.SemaphoreType.DMA for _ in range(6)],
        ],
        compiler_params=pltpu.CompilerParams(
            use_tc_tiling_on_sc=True, needs_layout_passes=False),
    )
    def seq_refresh(x_hbm, cols_hbm, o_hbm, idx_v, *rest):
        ina = rest[0:2]
        inb = rest[2:4]
        outb = rest[4:6]
        sa = rest[6:8]
        sb = rest[8:10]
        so = rest[10:12]
        wid = lax.axis_index("s") * _NC + lax.axis_index("c")

        pltpu.sync_copy(cols_hbm, idx_v)

        def decode(u):
            g = wid * units_w + u
            b = g // (n_units // B)
            rem = g % (n_units // B)
            ch = rem // (H // 2)
            hp = rem % (H // 2)
            return b, 2 * hp, ch * _CS

        def start_in(u, k):
            b, h0, c0 = decode(u)
            da = pltpu.async_copy(
                x_hbm.at[b, h0, pl.ds(c0, _CS), :], ina[k], sa[k])
            db = pltpu.async_copy(
                x_hbm.at[b, h0 + 1, pl.ds(c0, _CS), :], inb[k], sb[k])
            return da, db

        def wait_in(u, k):
            b, h0, c0 = decode(u)
            pltpu.make_async_copy(
                x_hbm.at[b, h0, pl.ds(c0, _CS), :], ina[k], sa[k]).wait()
            pltpu.make_async_copy(
                x_hbm.at[b, h0 + 1, pl.ds(c0, _CS), :], inb[k], sb[k]).wait()

        def out_dst(u):
            b, h0, c0 = decode(u)
            return o_hbm.at[b, pl.ds(c0, _CS), pl.ds(h0 * WW, 2 * WW)]

        def compact(k):
            def row(r, _):
                rv = jnp.full((_L,), r, jnp.int32)
                for j in range(n_j):
                    ca = idx_v[1, pl.ds(j * _L, _L)]
                    outb[k][r, pl.ds(j * _L, _L)] = plsc.load_gather(
                        ina[k], [rv, ca])
                    cb = idx_v[0, pl.ds(j * _L, _L)]
                    outb[k][r, pl.ds(WW + j * _L, _L)] = plsc.load_gather(
                        inb[k], [rv, cb])
                return _
            lax.fori_loop(0, _CS, row, None)

        start_in(0, 0)
        start_in(1, 1)

        def step(t, _):
            for k in (0, 1):
                u = 2 * t + k
                wait_in(u, k)

                @pl.when(t > 0)
                def _wait_out():
                    pltpu.make_async_copy(outb[k], out_dst(u - 2), so[k]).wait()

                compact(k)
                pltpu.async_copy(outb[k], out_dst(u), so[k])

                @pl.when(t < units_w // 2 - 1)
                def _next_in():
                    start_in(u + 2, k)
            return _

        lax.fori_loop(0, units_w // 2, step, None)
        pltpu.make_async_copy(outb[0], out_dst(units_w - 2), so[0]).wait()
        pltpu.make_async_copy(outb[1], out_dst(units_w - 1), so[1]).wait()

    out_t = seq_refresh(xt, cols_tab)
    return out_t.transpose(0, 2, 1)


# speedup vs baseline: 2.0977x; 1.6244x over previous
import functools

import jax
import jax.numpy as jnp
from jax import lax
from jax.experimental import pallas as pl
from jax.experimental.pallas import tpu as pltpu
from jax.experimental.pallas import tpu_sc as plsc

_NC, _NS = 2, 16
_NW = _NC * _NS
_CS = 48
_L = 16
_OPAD = 512


def kernel(inputs):
    B, H, W, C = inputs.shape
    WW = W // 2
    xt = inputs.transpose(0, 1, 3, 2)
    n_units = B * (H // 2) * (C // _CS)
    units_w = n_units // _NW
    n_j = W // _L

    mesh = plsc.VectorSubcoreMesh(
        core_axis_name="c", subcore_axis_name="s",
        num_cores=_NC, num_subcores=_NS,
    )

    @functools.partial(
        pl.kernel,
        out_type=jax.ShapeDtypeStruct((B, C, H * WW), inputs.dtype),
        mesh=mesh,
        scratch_types=[
            *[pltpu.VMEM((_CS, W), inputs.dtype) for _ in range(4)],
            *[pltpu.VMEM((_CS, _OPAD), inputs.dtype) for _ in range(2)],
            *[pltpu.SemaphoreType.DMA for _ in range(6)],
        ],
        compiler_params=pltpu.CompilerParams(
            use_tc_tiling_on_sc=True, needs_layout_passes=False),
    )
    def seq_refresh(x_hbm, o_hbm, *rest):
        ina = rest[0:2]
        inb = rest[2:4]
        outb = rest[4:6]
        sa = rest[6:8]
        sb = rest[8:10]
        so = rest[10:12]
        wid = lax.axis_index("s") * _NC + lax.axis_index("c")

        def decode(u):
            g = wid * units_w + u
            b = g // (n_units // B)
            rem = g % (n_units // B)
            ch = rem // (H // 2)
            hp = rem % (H // 2)
            return b, 2 * hp, ch * _CS

        def start_in(u, k):
            b, h0, c0 = decode(u)
            pltpu.async_copy(x_hbm.at[b, h0, pl.ds(c0, _CS), :], ina[k], sa[k])
            pltpu.async_copy(x_hbm.at[b, h0 + 1, pl.ds(c0, _CS), :], inb[k], sb[k])

        def wait_in(u, k):
            b, h0, c0 = decode(u)
            pltpu.make_async_copy(
                x_hbm.at[b, h0, pl.ds(c0, _CS), :], ina[k], sa[k]).wait()
            pltpu.make_async_copy(
                x_hbm.at[b, h0 + 1, pl.ds(c0, _CS), :], inb[k], sb[k]).wait()

        def out_dst(u):
            b, h0, c0 = decode(u)
            return o_hbm.at[b, pl.ds(c0, _CS), pl.ds(h0 * WW, 2 * WW)]

        lane_par = lax.iota(jnp.int32, _L) % 2
        m_odd = lane_par == 1
        m_even = lane_par == 0

        def compact(k):
            def row(r, _):
                for j in range(n_j):
                    va = ina[k][r, pl.ds(_L * j, _L)]
                    plsc.store_compressed(
                        outb[k].at[r, pl.ds(8 * j, _L)], va, mask=m_odd)
                    vb = inb[k][r, pl.ds(_L * j, _L)]
                    plsc.store_compressed(
                        outb[k].at[r, pl.ds(WW + 8 * j, _L)], vb, mask=m_even)
                return _
            lax.fori_loop(0, _CS, row, None)

        start_in(0, 0)
        start_in(1, 1)

        def step(t, _):
            for k in (0, 1):
                u = 2 * t + k
                wait_in(u, k)

                @pl.when(t > 0)
                def _wait_out():
                    pltpu.make_async_copy(outb[k].at[:, pl.ds(0, W)],
                                          out_dst(u - 2), so[k]).wait()

                compact(k)
                pltpu.async_copy(outb[k].at[:, pl.ds(0, W)], out_dst(u), so[k])

                @pl.when(t < units_w // 2 - 1)
                def _next_in():
                    start_in(u + 2, k)
            return _

        lax.fori_loop(0, units_w // 2, step, None)
        pltpu.make_async_copy(outb[0].at[:, pl.ds(0, W)],
                              out_dst(units_w - 2), so[0]).wait()
        pltpu.make_async_copy(outb[1].at[:, pl.ds(0, W)],
                              out_dst(units_w - 1), so[1]).wait()

    out_t = seq_refresh(xt)
    return out_t.transpose(0, 2, 1)


# speedup vs baseline: 5.8513x; 2.7894x over previous
import functools

import jax
import jax.numpy as jnp
from jax import lax
from jax.experimental import pallas as pl
from jax.experimental.pallas import tpu as pltpu
from jax.experimental.pallas import tpu_sc as plsc

_NC, _NS = 2, 16
_NW = _NC * _NS
_CS = 48
_L = 16
_OPAD = 512


def kernel(inputs):
    B, H, W, C = inputs.shape
    WW = W // 2
    xt = inputs.transpose(0, 1, 3, 2)
    n_units = B * (H // 2) * (C // _CS)
    units_w = n_units // _NW
    n_j = W // _L

    mesh = plsc.VectorSubcoreMesh(
        core_axis_name="c", subcore_axis_name="s",
        num_cores=_NC, num_subcores=_NS,
    )

    @functools.partial(
        pl.kernel,
        out_type=jax.ShapeDtypeStruct((B, C, H * WW), inputs.dtype),
        mesh=mesh,
        scratch_types=[
            *[pltpu.VMEM((_CS, W), inputs.dtype) for _ in range(4)],
            *[pltpu.VMEM((_CS, _OPAD), inputs.dtype) for _ in range(2)],
            *[pltpu.SemaphoreType.DMA for _ in range(6)],
        ],
        compiler_params=pltpu.CompilerParams(
            use_tc_tiling_on_sc=True, needs_layout_passes=False),
    )
    def seq_refresh(x_hbm, o_hbm, *rest):
        ina = rest[0:2]
        inb = rest[2:4]
        outb = rest[4:6]
        sa = rest[6:8]
        sb = rest[8:10]
        so = rest[10:12]
        wid = lax.axis_index("s") * _NC + lax.axis_index("c")

        def decode(u):
            g = wid * units_w + u
            b = g // (n_units // B)
            rem = g % (n_units // B)
            ch = rem // (H // 2)
            hp = rem % (H // 2)
            return b, 2 * hp, ch * _CS

        def start_in(u, k):
            b, h0, c0 = decode(u)
            pltpu.async_copy(x_hbm.at[b, h0, pl.ds(c0, _CS), :], ina[k], sa[k])
            pltpu.async_copy(x_hbm.at[b, h0 + 1, pl.ds(c0, _CS), :], inb[k], sb[k])

        def wait_in(u, k):
            b, h0, c0 = decode(u)
            pltpu.make_async_copy(
                x_hbm.at[b, h0, pl.ds(c0, _CS), :], ina[k], sa[k]).wait()
            pltpu.make_async_copy(
                x_hbm.at[b, h0 + 1, pl.ds(c0, _CS), :], inb[k], sb[k]).wait()

        def out_dst(u):
            b, h0, c0 = decode(u)
            return o_hbm.at[b, pl.ds(c0, _CS), pl.ds(h0 * WW, 2 * WW)]

        lane_par = lax.iota(jnp.int32, _L) % 2
        m_odd = lane_par == 1
        m_even = lane_par == 0

        def compact(k):
            @plsc.parallel_loop(0, _CS, unroll=2)
            def row(r):
                for j in range(n_j):
                    va = ina[k][r, pl.ds(_L * j, _L)]
                    plsc.store_compressed(
                        outb[k].at[r, pl.ds(8 * j, _L)], va, mask=m_odd)
                    vb = inb[k][r, pl.ds(_L * j, _L)]
                    plsc.store_compressed(
                        outb[k].at[r, pl.ds(WW + 8 * j, _L)], vb, mask=m_even)

        start_in(0, 0)
        start_in(1, 1)

        def step(t, _):
            for k in (0, 1):
                u = 2 * t + k
                wait_in(u, k)

                @pl.when(t > 0)
                def _wait_out():
                    pltpu.make_async_copy(outb[k].at[:, pl.ds(0, W)],
                                          out_dst(u - 2), so[k]).wait()

                compact(k)
                pltpu.async_copy(outb[k].at[:, pl.ds(0, W)], out_dst(u), so[k])

                @pl.when(t < units_w // 2 - 1)
                def _next_in():
                    start_in(u + 2, k)
            return _

        lax.fori_loop(0, units_w // 2, step, None)
        pltpu.make_async_copy(outb[0].at[:, pl.ds(0, W)],
                              out_dst(units_w - 2), so[0]).wait()
        pltpu.make_async_copy(outb[1].at[:, pl.ds(0, W)],
                              out_dst(units_w - 1), so[1]).wait()

    out_t = seq_refresh(xt)
    return out_t.transpose(0, 2, 1)
